# Initial kernel scaffold; baseline (speedup 1.0000x reference)
import jax
import jax.numpy as jnp
from jax.experimental import pallas as pl


def kernel(x, edge_index, batch, W_rel0, b_rel0, W_root0, W_rel1, b_rel1, W_root1, W1, b1, gamma, beta, W2, b2):
    raise NotImplementedError("write your pallas kernel here")


# speedup vs baseline: 7.7645x; 7.7645x over previous
import functools

import jax
import jax.numpy as jnp
from jax import lax
from jax.experimental import pallas as pl
from jax.experimental.pallas import tpu as pltpu
from jax.experimental.pallas import tpu_sc as plsc

N = 10000
E = 320000
D = 128
H = 128
G = 64
OUT = 1

NC = 2
NS = 16
NW = NC * NS
EPW = E // NW
K = 125
NCHUNK = EPW // K
WCHUNK = 80
NWCHUNK = N // WCHUNK
WPT = -(-NWCHUNK // NS)

BLK = 1000
NBLK = N // BLK


def _segsum_sc(table, src2d, dst2d, zeros):
  mesh = plsc.VectorSubcoreMesh(core_axis_name="c", subcore_axis_name="s")

  @functools.partial(
      pl.kernel,
      out_type=jax.ShapeDtypeStruct((NC * N, D), jnp.float32),
      mesh=mesh,
      scratch_types=[
          pltpu.VMEM((NCHUNK, K), jnp.int32),
          pltpu.VMEM((NCHUNK, K), jnp.int32),
          pltpu.VMEM((K, D), jnp.float32),
          pltpu.VMEM((WCHUNK, D), jnp.float32),
          pltpu.VMEM_SHARED((N, D), jnp.float32),
          pltpu.SemaphoreType.DMA,
      ],
  )
  def k(table_h, src_h, dst_h, zeros_h, out_h,
        src_v, dst_v, rows_v, wb_v, acc_sh, sem):
    c = lax.axis_index("c")
    s = lax.axis_index("s")
    wid = s * NC + c

    @pl.when(s == 0)
    def _():
      pltpu.sync_copy(zeros_h, acc_sh)

    pltpu.sync_copy(src_h.at[pl.ds(wid * NCHUNK, NCHUNK)], src_v)
    pltpu.sync_copy(dst_h.at[pl.ds(wid * NCHUNK, NCHUNK)], dst_v)
    plsc.subcore_barrier()

    def body(j, carry):
      pltpu.async_copy(table_h.at[src_v.at[j]], rows_v, sem).wait()
      pltpu.sync_copy(rows_v, acc_sh.at[dst_v.at[j]], add=True)
      return carry

    lax.fori_loop(0, NCHUNK, body, 0)
    plsc.subcore_barrier()

    def wbody(j, carry):
      m = s + j * NS

      @pl.when(m < NWCHUNK)
      def _():
        r0 = m * WCHUNK
        pltpu.sync_copy(acc_sh.at[pl.ds(r0, WCHUNK)], wb_v)
        pltpu.sync_copy(wb_v, out_h.at[pl.ds(c * N + r0, WCHUNK)])

      return carry

    lax.fori_loop(0, WPT, wbody, 0)

  return k(table, src2d, dst2d, zeros)


def _gconv_dense(parts, x, W_rel, W_root, b_rel):
  def body(p0_ref, p1_ref, x_ref, wr_ref, wt_ref, b_ref, o_ref):
    agg = p0_ref[...] + p1_ref[...]
    acc = jnp.dot(agg, wr_ref[...], preferred_element_type=jnp.float32)
    acc += jnp.dot(x_ref[...], wt_ref[...], preferred_element_type=jnp.float32)
    o_ref[...] = jnp.maximum(acc + b_ref[...], 0.0)

  return pl.pallas_call(
      body,
      grid=(NBLK,),
      in_specs=[
          pl.BlockSpec((BLK, D), lambda i: (i, 0)),
          pl.BlockSpec((BLK, D), lambda i: (i + NBLK, 0)),
          pl.BlockSpec((BLK, D), lambda i: (i, 0)),
          pl.BlockSpec((D, H), lambda i: (0, 0)),
          pl.BlockSpec((D, H), lambda i: (0, 0)),
          pl.BlockSpec((1, H), lambda i: (0, 0)),
      ],
      out_specs=pl.BlockSpec((BLK, H), lambda i: (i, 0)),
      out_shape=jax.ShapeDtypeStruct((N, H), jnp.float32),
  )(parts, parts, x, W_rel, W_root, b_rel.reshape(1, H))


def _final(parts, h1, W_rel, W_root, b_rel, batch3, W1, b1, gamma, beta,
           W2p, b2p):
  def body(p0_ref, p1_ref, h1_ref, wr_ref, wt_ref, br_ref, bt_ref,
           w1_ref, b1_ref, g_ref, be_ref, w2_ref, b2_ref, o_ref, pacc):
    i = pl.program_id(0)

    @pl.when(i == 0)
    def _():
      pacc[...] = jnp.zeros_like(pacc)

    agg = p0_ref[...] + p1_ref[...]
    acc = jnp.dot(agg, wr_ref[...], preferred_element_type=jnp.float32)
    acc += jnp.dot(h1_ref[...], wt_ref[...], preferred_element_type=jnp.float32)
    h2 = jnp.maximum(acc + br_ref[...], 0.0)

    b = bt_ref[0]
    gid = lax.broadcasted_iota(jnp.int32, (G, BLK), 0)
    onehot = (gid == b).astype(jnp.float32)
    pacc[...] += jnp.dot(onehot, h2, preferred_element_type=jnp.float32)

    @pl.when(i == NBLK - 1)
    def _():
      z = jnp.dot(pacc[...], w1_ref[...],
                  preferred_element_type=jnp.float32) + b1_ref[...]
      mu = jnp.mean(z, axis=0, keepdims=True)
      var = jnp.mean((z - mu) * (z - mu), axis=0, keepdims=True)
      zn = (z - mu) * lax.rsqrt(var + 1e-5) * g_ref[...] + be_ref[...]
      zn = jnp.maximum(zn, 0.0)
      logits = jnp.dot(zn, w2_ref[...],
                       preferred_element_type=jnp.float32) + b2_ref[...]
      o_ref[...] = jax.nn.sigmoid(logits)

  H4 = 4 * H
  return pl.pallas_call(
      body,
      grid=(NBLK,),
      in_specs=[
          pl.BlockSpec((BLK, D), lambda i: (i, 0)),
          pl.BlockSpec((BLK, D), lambda i: (i + NBLK, 0)),
          pl.BlockSpec((BLK, H), lambda i: (i, 0)),
          pl.BlockSpec((H, H), lambda i: (0, 0)),
          pl.BlockSpec((H, H), lambda i: (0, 0)),
          pl.BlockSpec((1, H), lambda i: (0, 0)),
          pl.BlockSpec((1, 1, BLK), lambda i: (i, 0, 0)),
          pl.BlockSpec((H, H4), lambda i: (0, 0)),
          pl.BlockSpec((1, H4), lambda i: (0, 0)),
          pl.BlockSpec((1, H4), lambda i: (0, 0)),
          pl.BlockSpec((1, H4), lambda i: (0, 0)),
          pl.BlockSpec((H4, 128), lambda i: (0, 0)),
          pl.BlockSpec((1, 128), lambda i: (0, 0)),
      ],
      out_specs=pl.BlockSpec((G, 128), lambda i: (0, 0)),
      out_shape=jax.ShapeDtypeStruct((G, 128), jnp.float32),
      scratch_shapes=[pltpu.VMEM((G, H), jnp.float32)],
  )(parts, parts, h1, W_rel, W_root, b_rel.reshape(1, H),
    batch3, W1, b1.reshape(1, H4), gamma.reshape(1, H4),
    beta.reshape(1, H4), W2p, b2p)


def kernel(x, edge_index, batch, W_rel0, b_rel0, W_root0, W_rel1, b_rel1,
           W_root1, W1, b1, gamma, beta, W2, b2):
  src2 = edge_index[0].reshape(E // K, K)
  dst2 = edge_index[1].reshape(E // K, K)
  zeros = jnp.zeros((N, D), jnp.float32)
  batch3 = batch.reshape(NBLK, 1, BLK)
  W2p = jnp.pad(W2, ((0, 0), (0, 128 - OUT)))
  b2p = jnp.pad(b2, (0, 128 - OUT)).reshape(1, 128)

  parts0 = _segsum_sc(x, src2, dst2, zeros)
  h1 = _gconv_dense(parts0, x, W_rel0, W_root0, b_rel0)
  parts1 = _segsum_sc(h1, src2, dst2, zeros)
  out = _final(parts1, h1, W_rel1, W_root1, b_rel1, batch3, W1, b1,
               gamma, beta, W2p, b2p)
  return out[:, :OUT]


# speedup vs baseline: 10.5330x; 1.3566x over previous
import functools

import jax
import jax.numpy as jnp
from jax import lax
from jax.experimental import pallas as pl
from jax.experimental.pallas import tpu as pltpu
from jax.experimental.pallas import tpu_sc as plsc

N = 10000
E = 320000
D = 128
H = 128
G = 64
OUT = 1

NC = 2
NS = 16
NW = NC * NS
EPW = E // NW
K = 125
NCHUNK = EPW // K
NH = 2
CPH = NCHUNK // NH
WCHUNK = 80
NWCHUNK = N // WCHUNK
WPT = -(-NWCHUNK // NS)

BLK = 1000
NBLK = N // BLK


def _segsum_sc(table, src2d, dst2d, zeros):
  mesh = plsc.VectorSubcoreMesh(core_axis_name="c", subcore_axis_name="s")

  @functools.partial(
      pl.kernel,
      out_type=jax.ShapeDtypeStruct((NC * N, D), jnp.float32),
      mesh=mesh,
      scratch_types=[
          pltpu.VMEM((CPH, K), jnp.int32),
          pltpu.VMEM((CPH, K), jnp.int32),
          pltpu.VMEM((K, D), jnp.float32),
          pltpu.VMEM((K, D), jnp.float32),
          pltpu.VMEM_SHARED((N, D), jnp.float32),
          pltpu.SemaphoreType.DMA,
          pltpu.SemaphoreType.DMA,
      ],
  )
  def k(table_h, src_h, dst_h, zeros_h, out_h,
        src_v, dst_v, rows0_v, rows1_v, acc_sh, gsem0, gsem1):
    c = lax.axis_index("c")
    s = lax.axis_index("s")
    wid = s * NC + c
    rows = (rows0_v, rows1_v)
    gsem = (gsem0, gsem1)

    def zbody(j, carry):
      m = s + j * NS

      @pl.when(m < NWCHUNK)
      def _():
        pltpu.sync_copy(zeros_h, acc_sh.at[pl.ds(m * WCHUNK, WCHUNK)])

      return carry

    lax.fori_loop(0, WPT, zbody, 0)
    plsc.subcore_barrier()

    for half in range(NH):
      base = wid * NCHUNK + half * CPH
      pltpu.sync_copy(src_h.at[pl.ds(base, CPH)], src_v)
      pltpu.sync_copy(dst_h.at[pl.ds(base, CPH)], dst_v)
      for b in range(2):
        pltpu.async_copy(table_h.at[src_v.at[b]], rows[b], gsem[b])

      def body(t, carry):
        for b in range(2):
          j = t * 2 + b
          pltpu.make_async_copy(table_h.at[src_v.at[j]], rows[b],
                                gsem[b]).wait()
          pltpu.sync_copy(rows[b], acc_sh.at[dst_v.at[j]], add=True)

          @pl.when(j < CPH - 2)
          def _():
            pltpu.async_copy(table_h.at[src_v.at[j + 2]], rows[b], gsem[b])

        return carry

      lax.fori_loop(0, CPH // 2, body, 0)

    plsc.subcore_barrier()

    def wbody(j, carry):
      m = s + j * NS

      @pl.when(m < NWCHUNK)
      def _():
        r0 = m * WCHUNK
        pltpu.sync_copy(acc_sh.at[pl.ds(r0, WCHUNK)],
                        out_h.at[pl.ds(c * N + r0, WCHUNK)])

      return carry

    lax.fori_loop(0, WPT, wbody, 0)

  return k(table, src2d, dst2d, zeros)


def _gconv_dense(parts, x, W_rel, W_root, b_rel):
  def body(p0_ref, p1_ref, x_ref, wr_ref, wt_ref, b_ref, o_ref):
    agg = p0_ref[...] + p1_ref[...]
    acc = jnp.dot(agg, wr_ref[...], preferred_element_type=jnp.float32)
    acc += jnp.dot(x_ref[...], wt_ref[...], preferred_element_type=jnp.float32)
    o_ref[...] = jnp.maximum(acc + b_ref[...], 0.0)

  return pl.pallas_call(
      body,
      grid=(NBLK,),
      in_specs=[
          pl.BlockSpec((BLK, D), lambda i: (i, 0)),
          pl.BlockSpec((BLK, D), lambda i: (i + NBLK, 0)),
          pl.BlockSpec((BLK, D), lambda i: (i, 0)),
          pl.BlockSpec((D, H), lambda i: (0, 0)),
          pl.BlockSpec((D, H), lambda i: (0, 0)),
          pl.BlockSpec((1, H), lambda i: (0, 0)),
      ],
      out_specs=pl.BlockSpec((BLK, H), lambda i: (i, 0)),
      out_shape=jax.ShapeDtypeStruct((N, H), jnp.float32),
  )(parts, parts, x, W_rel, W_root, b_rel.reshape(1, H))


def _final(parts, h1, W_rel, W_root, b_rel, batch3, W1, b1, gamma, beta,
           W2p, b2p):
  def body(p0_ref, p1_ref, h1_ref, wr_ref, wt_ref, br_ref, bt_ref,
           w1_ref, b1_ref, g_ref, be_ref, w2_ref, b2_ref, o_ref, pacc):
    i = pl.program_id(0)

    @pl.when(i == 0)
    def _():
      pacc[...] = jnp.zeros_like(pacc)

    agg = p0_ref[...] + p1_ref[...]
    acc = jnp.dot(agg, wr_ref[...], preferred_element_type=jnp.float32)
    acc += jnp.dot(h1_ref[...], wt_ref[...], preferred_element_type=jnp.float32)
    h2 = jnp.maximum(acc + br_ref[...], 0.0)

    b = bt_ref[0]
    gid = lax.broadcasted_iota(jnp.int32, (G, BLK), 0)
    onehot = (gid == b).astype(jnp.float32)
    pacc[...] += jnp.dot(onehot, h2, preferred_element_type=jnp.float32)

    @pl.when(i == NBLK - 1)
    def _():
      z = jnp.dot(pacc[...], w1_ref[...],
                  preferred_element_type=jnp.float32) + b1_ref[...]
      mu = jnp.mean(z, axis=0, keepdims=True)
      var = jnp.mean((z - mu) * (z - mu), axis=0, keepdims=True)
      zn = (z - mu) * lax.rsqrt(var + 1e-5) * g_ref[...] + be_ref[...]
      zn = jnp.maximum(zn, 0.0)
      logits = jnp.dot(zn, w2_ref[...],
                       preferred_element_type=jnp.float32) + b2_ref[...]
      o_ref[...] = jax.nn.sigmoid(logits)

  H4 = 4 * H
  return pl.pallas_call(
      body,
      grid=(NBLK,),
      in_specs=[
          pl.BlockSpec((BLK, D), lambda i: (i, 0)),
          pl.BlockSpec((BLK, D), lambda i: (i + NBLK, 0)),
          pl.BlockSpec((BLK, H), lambda i: (i, 0)),
          pl.BlockSpec((H, H), lambda i: (0, 0)),
          pl.BlockSpec((H, H), lambda i: (0, 0)),
          pl.BlockSpec((1, H), lambda i: (0, 0)),
          pl.BlockSpec((1, 1, BLK), lambda i: (i, 0, 0)),
          pl.BlockSpec((H, H4), lambda i: (0, 0)),
          pl.BlockSpec((1, H4), lambda i: (0, 0)),
          pl.BlockSpec((1, H4), lambda i: (0, 0)),
          pl.BlockSpec((1, H4), lambda i: (0, 0)),
          pl.BlockSpec((H4, 128), lambda i: (0, 0)),
          pl.BlockSpec((1, 128), lambda i: (0, 0)),
      ],
      out_specs=pl.BlockSpec((G, 128), lambda i: (0, 0)),
      out_shape=jax.ShapeDtypeStruct((G, 128), jnp.float32),
      scratch_shapes=[pltpu.VMEM((G, H), jnp.float32)],
  )(parts, parts, h1, W_rel, W_root, b_rel.reshape(1, H),
    batch3, W1, b1.reshape(1, H4), gamma.reshape(1, H4),
    beta.reshape(1, H4), W2p, b2p)


def kernel(x, edge_index, batch, W_rel0, b_rel0, W_root0, W_rel1, b_rel1,
           W_root1, W1, b1, gamma, beta, W2, b2):
  src2 = edge_index[0].reshape(E // K, K)
  dst2 = edge_index[1].reshape(E // K, K)
  zeros = jnp.zeros((WCHUNK, D), jnp.float32)
  batch3 = batch.reshape(NBLK, 1, BLK)
  W2p = jnp.pad(W2, ((0, 0), (0, 128 - OUT)))
  b2p = jnp.pad(b2, (0, 128 - OUT)).reshape(1, 128)

  parts0 = _segsum_sc(x, src2, dst2, zeros)
  h1 = _gconv_dense(parts0, x, W_rel0, W_root0, b_rel0)
  parts1 = _segsum_sc(h1, src2, dst2, zeros)
  out = _final(parts1, h1, W_rel1, W_root1, b_rel1, batch3, W1, b1,
               gamma, beta, W2p, b2p)
  return out[:, :OUT]


# speedup vs baseline: 11.2232x; 1.0655x over previous
import functools

import jax
import jax.numpy as jnp
from jax import lax
from jax.experimental import pallas as pl
from jax.experimental.pallas import tpu as pltpu
from jax.experimental.pallas import tpu_sc as plsc

N = 10000
E = 320000
D = 128
H = 128
G = 64
OUT = 1

NC = 2
NS = 16
NW = NC * NS
EPW = E // NW
K = 80
NCHUNK = EPW // K
NBUF = 4
WCHUNK = 80
NWCHUNK = N // WCHUNK
WPT = -(-NWCHUNK // NS)

BLK = 1000
NBLK = N // BLK


def _segsum_sc(table, idx3, zeros):
  mesh = plsc.VectorSubcoreMesh(core_axis_name="c", subcore_axis_name="s")

  @functools.partial(
      pl.kernel,
      out_type=jax.ShapeDtypeStruct((NC * N, D), jnp.float32),
      mesh=mesh,
      scratch_types=[
          pltpu.VMEM((2 * NBUF, K), jnp.int32),
          pltpu.VMEM((K, D), jnp.float32),
          pltpu.VMEM((K, D), jnp.float32),
          pltpu.VMEM((K, D), jnp.float32),
          pltpu.VMEM((K, D), jnp.float32),
          pltpu.VMEM_SHARED((N, D), jnp.float32),
          [pltpu.SemaphoreType.DMA] * NBUF,
          [pltpu.SemaphoreType.DMA] * NBUF,
          [pltpu.SemaphoreType.DMA] * NBUF,
      ],
  )
  def k(table_h, idx_h, zeros_h, out_h,
        idx_v, rows0_v, rows1_v, rows2_v, rows3_v, acc_sh,
        isem, gsem, ssem):
    c = lax.axis_index("c")
    s = lax.axis_index("s")
    wid = s * NC + c
    rows = (rows0_v, rows1_v, rows2_v, rows3_v)
    base = wid * NCHUNK

    def idx_start(i, b):
      pltpu.async_copy(idx_h.at[base + i], idx_v.at[pl.ds(2 * b, 2)], isem[b])

    def idx_wait(i, b):
      pltpu.make_async_copy(idx_h.at[base + i], idx_v.at[pl.ds(2 * b, 2)],
                            isem[b]).wait()

    def gather_start(b):
      pltpu.async_copy(table_h.at[idx_v.at[2 * b]], rows[b], gsem[b])

    def gather_wait(b):
      pltpu.make_async_copy(table_h.at[idx_v.at[2 * b]], rows[b],
                            gsem[b]).wait()

    def scatter_start(b):
      pltpu.async_copy(rows[b], acc_sh.at[idx_v.at[2 * b + 1]], ssem[b],
                       add=True)

    def scatter_wait(b):
      pltpu.make_async_copy(rows[b], acc_sh.at[idx_v.at[2 * b + 1]],
                            ssem[b]).wait()

    def zbody(j, carry):
      m = s + j * NS

      @pl.when(m < NWCHUNK)
      def _():
        pltpu.sync_copy(zeros_h, acc_sh.at[pl.ds(m * WCHUNK, WCHUNK)])

      return carry

    lax.fori_loop(0, WPT, zbody, 0)
    plsc.subcore_barrier()

    idx_start(0, 0)
    idx_start(1, 1)
    idx_wait(0, 0)
    gather_start(0)

    def body(t, carry):
      for u in range(4):
        i = t * 4 + u
        b1 = (u + 1) % 4
        b2 = (u + 2) % 4

        @pl.when(i >= 2)
        def _():
          scatter_wait(b2)

        @pl.when(i + 2 < NCHUNK)
        def _():
          idx_start(i + 2, b2)

        idx_wait(i + 1, b1)
        gather_start(b1)
        gather_wait(u)
        scatter_start(u)
      return carry

    lax.fori_loop(0, (NCHUNK - 1) // 4, body, 0)

    scatter_wait(2)
    gather_wait(0)
    scatter_start(0)
    scatter_wait(3)
    scatter_wait(0)
    plsc.subcore_barrier()

    def wbody(j, carry):
      m = s + j * NS

      @pl.when(m < NWCHUNK)
      def _():
        r0 = m * WCHUNK
        pltpu.sync_copy(acc_sh.at[pl.ds(r0, WCHUNK)],
                        out_h.at[pl.ds(c * N + r0, WCHUNK)])

      return carry

    lax.fori_loop(0, WPT, wbody, 0)

  return k(table, idx3, zeros)


def _gconv_dense(parts, x, W_rel, W_root, b_rel):
  def body(p0_ref, p1_ref, x_ref, wr_ref, wt_ref, b_ref, o_ref):
    agg = p0_ref[...] + p1_ref[...]
    acc = jnp.dot(agg, wr_ref[...], preferred_element_type=jnp.float32)
    acc += jnp.dot(x_ref[...], wt_ref[...], preferred_element_type=jnp.float32)
    o_ref[...] = jnp.maximum(acc + b_ref[...], 0.0)

  return pl.pallas_call(
      body,
      grid=(NBLK,),
      in_specs=[
          pl.BlockSpec((BLK, D), lambda i: (i, 0)),
          pl.BlockSpec((BLK, D), lambda i: (i + NBLK, 0)),
          pl.BlockSpec((BLK, D), lambda i: (i, 0)),
          pl.BlockSpec((D, H), lambda i: (0, 0)),
          pl.BlockSpec((D, H), lambda i: (0, 0)),
          pl.BlockSpec((1, H), lambda i: (0, 0)),
      ],
      out_specs=pl.BlockSpec((BLK, H), lambda i: (i, 0)),
      out_shape=jax.ShapeDtypeStruct((N, H), jnp.float32),
  )(parts, parts, x, W_rel, W_root, b_rel.reshape(1, H))


def _final(parts, h1, W_rel, W_root, b_rel, batch3, W1, b1, gamma, beta,
           W2p, b2p):
  def body(p0_ref, p1_ref, h1_ref, wr_ref, wt_ref, br_ref, bt_ref,
           w1_ref, b1_ref, g_ref, be_ref, w2_ref, b2_ref, o_ref, pacc):
    i = pl.program_id(0)

    @pl.when(i == 0)
    def _():
      pacc[...] = jnp.zeros_like(pacc)

    agg = p0_ref[...] + p1_ref[...]
    acc = jnp.dot(agg, wr_ref[...], preferred_element_type=jnp.float32)
    acc += jnp.dot(h1_ref[...], wt_ref[...], preferred_element_type=jnp.float32)
    h2 = jnp.maximum(acc + br_ref[...], 0.0)

    b = bt_ref[0]
    gid = lax.broadcasted_iota(jnp.int32, (G, BLK), 0)
    onehot = (gid == b).astype(jnp.float32)
    pacc[...] += jnp.dot(onehot, h2, preferred_element_type=jnp.float32)

    @pl.when(i == NBLK - 1)
    def _():
      z = jnp.dot(pacc[...], w1_ref[...],
                  preferred_element_type=jnp.float32) + b1_ref[...]
      mu = jnp.mean(z, axis=0, keepdims=True)
      var = jnp.mean((z - mu) * (z - mu), axis=0, keepdims=True)
      zn = (z - mu) * lax.rsqrt(var + 1e-5) * g_ref[...] + be_ref[...]
      zn = jnp.maximum(zn, 0.0)
      logits = jnp.dot(zn, w2_ref[...],
                       preferred_element_type=jnp.float32) + b2_ref[...]
      o_ref[...] = jax.nn.sigmoid(logits)

  H4 = 4 * H
  return pl.pallas_call(
      body,
      grid=(NBLK,),
      in_specs=[
          pl.BlockSpec((BLK, D), lambda i: (i, 0)),
          pl.BlockSpec((BLK, D), lambda i: (i + NBLK, 0)),
          pl.BlockSpec((BLK, H), lambda i: (i, 0)),
          pl.BlockSpec((H, H), lambda i: (0, 0)),
          pl.BlockSpec((H, H), lambda i: (0, 0)),
          pl.BlockSpec((1, H), lambda i: (0, 0)),
          pl.BlockSpec((1, 1, BLK), lambda i: (i, 0, 0)),
          pl.BlockSpec((H, H4), lambda i: (0, 0)),
          pl.BlockSpec((1, H4), lambda i: (0, 0)),
          pl.BlockSpec((1, H4), lambda i: (0, 0)),
          pl.BlockSpec((1, H4), lambda i: (0, 0)),
          pl.BlockSpec((H4, 128), lambda i: (0, 0)),
          pl.BlockSpec((1, 128), lambda i: (0, 0)),
      ],
      out_specs=pl.BlockSpec((G, 128), lambda i: (0, 0)),
      out_shape=jax.ShapeDtypeStruct((G, 128), jnp.float32),
      scratch_shapes=[pltpu.VMEM((G, H), jnp.float32)],
  )(parts, parts, h1, W_rel, W_root, b_rel.reshape(1, H),
    batch3, W1, b1.reshape(1, H4), gamma.reshape(1, H4),
    beta.reshape(1, H4), W2p, b2p)


def kernel(x, edge_index, batch, W_rel0, b_rel0, W_root0, W_rel1, b_rel1,
           W_root1, W1, b1, gamma, beta, W2, b2):
  idx3 = jnp.stack([edge_index[0].reshape(E // K, K),
                    edge_index[1].reshape(E // K, K)], axis=1)
  zeros = jnp.zeros((WCHUNK, D), jnp.float32)
  batch3 = batch.reshape(NBLK, 1, BLK)
  W2p = jnp.pad(W2, ((0, 0), (0, 128 - OUT)))
  b2p = jnp.pad(b2, (0, 128 - OUT)).reshape(1, 128)

  parts0 = _segsum_sc(x, idx3, zeros)
  h1 = _gconv_dense(parts0, x, W_rel0, W_root0, b_rel0)
  parts1 = _segsum_sc(h1, idx3, zeros)
  out = _final(parts1, h1, W_rel1, W_root1, b_rel1, batch3, W1, b1,
               gamma, beta, W2p, b2p)
  return out[:, :OUT]
